# baseline (speedup 1.0000x reference)
import functools

import jax
import jax.numpy as jnp
from jax import lax
from jax.experimental import pallas as pl
from jax.experimental.pallas import tpu as pltpu
from jax.experimental.pallas import tpu_sc as plsc

B = 16384
EMBED = 64
FEAT = 16
NC = 2
NS = 16
NW = NC * NS
BPW = B // NW
CH = 128
NCH = BPW // CH


def _gather_body(uid_hbm, iid_hbm, ut_hbm, it_hbm, ft_hbm,
                 ue_hbm, ie_hbm, fe_hbm,
                 idx_u, idx_i, rows_u, rows_i, rows_f, sem):
    c = lax.axis_index("c")
    s = lax.axis_index("s")
    wid = s * NC + c
    pltpu.sync_copy(uid_hbm.at[pl.ds(wid * NCH, NCH)], idx_u)
    pltpu.sync_copy(iid_hbm.at[pl.ds(wid * NCH, NCH)], idx_i)
    copies = []
    for j in range(NCH):
        copies.append(pltpu.async_copy(
            ut_hbm.at[idx_u.at[j]], rows_u.at[pl.ds(j * CH, CH)], sem))
        copies.append(pltpu.async_copy(
            it_hbm.at[idx_i.at[j]], rows_i.at[pl.ds(j * CH, CH)], sem))
        copies.append(pltpu.async_copy(
            ft_hbm.at[idx_i.at[j]], rows_f.at[pl.ds(j * CH, CH)], sem))
    for cp in copies:
        cp.wait()
    base = wid * BPW
    pltpu.sync_copy(rows_u, ue_hbm.at[pl.ds(base, BPW)])
    pltpu.sync_copy(rows_i, ie_hbm.at[pl.ds(base, BPW)])
    pltpu.sync_copy(rows_f, fe_hbm.at[pl.ds(base, BPW)])


@functools.cache
def _gather():
    return pl.kernel(
        _gather_body,
        out_type=(
            jax.ShapeDtypeStruct((B, EMBED), jnp.float32),
            jax.ShapeDtypeStruct((B, EMBED), jnp.float32),
            jax.ShapeDtypeStruct((B, FEAT), jnp.float32),
        ),
        mesh=plsc.VectorSubcoreMesh(
            core_axis_name="c", subcore_axis_name="s",
            num_cores=NC, num_subcores=NS),
        scratch_types=[
            pltpu.VMEM((NCH, CH), jnp.int32),
            pltpu.VMEM((NCH, CH), jnp.int32),
            pltpu.VMEM((BPW, EMBED), jnp.float32),
            pltpu.VMEM((BPW, EMBED), jnp.float32),
            pltpu.VMEM((BPW, FEAT), jnp.float32),
            pltpu.SemaphoreType.DMA,
        ],
        compiler_params=pltpu.CompilerParams(use_tc_tiling_on_sc=False),
    )


BLK = 2048


def _mlp_body(ue, ie, fe, W1, b1, W2, b2, W3, b3, Ww, bw, out):
    ue_v = ue[...]
    ie_v = ie[...]
    fe_v = fe[...]
    dot = functools.partial(jnp.dot, preferred_element_type=jnp.float32)
    h1 = dot(ue_v, W1[:EMBED, :]) + dot(ie_v, W1[EMBED:2 * EMBED, :])
    h1 = h1 + dot(fe_v, W1[2 * EMBED:, :]) + b1[...]
    h1 = jnp.maximum(h1, 0.0)
    h2 = jnp.maximum(dot(h1, W2[...]) + b2[...], 0.0)
    deep = jnp.maximum(dot(h2, W3[...]) + b3[...], 0.0)
    wide = dot(ue_v, Ww[:EMBED, :]) + dot(ie_v, Ww[EMBED:2 * EMBED, :])
    wide = wide + dot(fe_v, Ww[2 * EMBED:, :]) + bw[...]
    out[...] = (deep + wide)[:, 0]


def _mlp(ue, ie, fe, W1, b1, W2, b2, W3, b3, Ww, bw):
    d_in = 2 * EMBED + FEAT
    grid = B // BLK
    rows = lambda i: (i, 0)
    full = lambda i: (0, 0)
    return pl.pallas_call(
        _mlp_body,
        grid=(grid,),
        in_specs=[
            pl.BlockSpec((BLK, EMBED), rows),
            pl.BlockSpec((BLK, EMBED), rows),
            pl.BlockSpec((BLK, FEAT), rows),
            pl.BlockSpec((d_in, 64), full),
            pl.BlockSpec((1, 64), full),
            pl.BlockSpec((64, 32), full),
            pl.BlockSpec((1, 32), full),
            pl.BlockSpec((32, 1), full),
            pl.BlockSpec((1, 1), full),
            pl.BlockSpec((d_in, 1), full),
            pl.BlockSpec((1, 1), full),
        ],
        out_specs=pl.BlockSpec((BLK,), lambda i: (i,)),
        out_shape=jax.ShapeDtypeStruct((B,), jnp.float32),
    )(ue, ie, fe, W1, b1, W2, b2, W3, b3, Ww, bw)


def kernel(user_id, item_id, user_table, item_table, feat_table,
           W1, b1, W2, b2, W3, b3, Ww, bw):
    uid = user_id.astype(jnp.int32).reshape(NW * NCH, CH)
    iid = item_id.astype(jnp.int32).reshape(NW * NCH, CH)
    ue, ie, fe = _gather()(uid, iid, user_table, item_table, feat_table)
    return _mlp(ue, ie, fe,
                W1, b1.reshape(1, 64), W2, b2.reshape(1, 32),
                W3, b3.reshape(1, 1), Ww, bw.reshape(1, 1))


# speedup vs baseline: 1.5792x; 1.5792x over previous
import functools

import jax
import jax.numpy as jnp
from jax import lax
from jax.experimental import pallas as pl
from jax.experimental.pallas import tpu as pltpu
from jax.experimental.pallas import tpu_sc as plsc

B = 16384
EMBED = 64
FEAT = 16
NC = 2
NS = 16
NW = NC * NS
BPW = B // NW
CH = 64
NCH = BPW // CH


def _gather_body(uid_hbm, iid_hbm, ut_hbm, it_hbm, ft_hbm,
                 ue_hbm, ie_hbm, fe_hbm,
                 idx_u, idx_i, buf_u, buf_i, buf_f, sem, osem):
    c = lax.axis_index("c")
    s = lax.axis_index("s")
    wid = s * NC + c
    base = wid * BPW
    pltpu.sync_copy(uid_hbm.at[pl.ds(base, BPW)], idx_u)
    pltpu.sync_copy(iid_hbm.at[pl.ds(base, BPW)], idx_i)

    def out_copies(k, p):
        ob = base + k * CH
        return (
            pltpu.make_async_copy(buf_u.at[p], ue_hbm.at[pl.ds(ob, CH)], osem),
            pltpu.make_async_copy(buf_i.at[p], ie_hbm.at[pl.ds(ob, CH)], osem),
            pltpu.make_async_copy(buf_f.at[p], fe_hbm.at[pl.ds(ob, CH)], osem),
        )

    def chunk(k, _):
        p = lax.rem(k, 2)
        descs = []
        for g in range(CH // 16):
            uvec = idx_u[pl.ds(k * CH + g * 16, 16)]
            ivec = idx_i[pl.ds(k * CH + g * 16, 16)]
            for l in range(16):
                r = g * 16 + l
                u = uvec[l]
                i = ivec[l]
                descs.append(pltpu.async_copy(
                    ut_hbm.at[pl.ds(u, 1), :], buf_u.at[p, pl.ds(r, 1), :], sem))
                descs.append(pltpu.async_copy(
                    it_hbm.at[pl.ds(i, 1), :], buf_i.at[p, pl.ds(r, 1), :], sem))
                descs.append(pltpu.async_copy(
                    ft_hbm.at[pl.ds(i, 1), :], buf_f.at[p, pl.ds(r, 1), :], sem))
        @pl.when(k >= 2)
        def _():
            for d in out_copies(k - 2, p):
                d.wait()
        for d in descs:
            d.wait()
        for d in out_copies(k, p):
            d.start()
        return ()

    lax.fori_loop(0, NCH, chunk, (), unroll=False)
    for k in (NCH - 2, NCH - 1):
        for d in out_copies(k, k % 2):
            d.wait()


@functools.cache
def _gather():
    return pl.kernel(
        _gather_body,
        out_type=(
            jax.ShapeDtypeStruct((B, EMBED), jnp.float32),
            jax.ShapeDtypeStruct((B, EMBED), jnp.float32),
            jax.ShapeDtypeStruct((B, FEAT), jnp.float32),
        ),
        mesh=plsc.VectorSubcoreMesh(
            core_axis_name="c", subcore_axis_name="s",
            num_cores=NC, num_subcores=NS),
        scratch_types=[
            pltpu.VMEM((BPW,), jnp.int32),
            pltpu.VMEM((BPW,), jnp.int32),
            pltpu.VMEM((2, CH, EMBED), jnp.float32),
            pltpu.VMEM((2, CH, EMBED), jnp.float32),
            pltpu.VMEM((2, CH, FEAT), jnp.float32),
            pltpu.SemaphoreType.DMA,
            pltpu.SemaphoreType.DMA,
        ],
        compiler_params=pltpu.CompilerParams(use_tc_tiling_on_sc=True),
    )


BLK = 2048


def _mlp_body(ue, ie, fe, W1, b1, W2, b2, W3, b3, Ww, bw, out):
    ue_v = ue[...]
    ie_v = ie[...]
    fe_v = fe[...]
    dot = functools.partial(jnp.dot, preferred_element_type=jnp.float32)
    h1 = dot(ue_v, W1[:EMBED, :]) + dot(ie_v, W1[EMBED:2 * EMBED, :])
    h1 = h1 + dot(fe_v, W1[2 * EMBED:, :]) + b1[...]
    h1 = jnp.maximum(h1, 0.0)
    h2 = jnp.maximum(dot(h1, W2[...]) + b2[...], 0.0)
    deep = jnp.maximum(dot(h2, W3[...]) + b3[...], 0.0)
    wide = dot(ue_v, Ww[:EMBED, :]) + dot(ie_v, Ww[EMBED:2 * EMBED, :])
    wide = wide + dot(fe_v, Ww[2 * EMBED:, :]) + bw[...]
    out[...] = (deep + wide)[:, 0]


def _mlp(ue, ie, fe, W1, b1, W2, b2, W3, b3, Ww, bw):
    d_in = 2 * EMBED + FEAT
    grid = B // BLK
    rows = lambda i: (i, 0)
    full = lambda i: (0, 0)
    return pl.pallas_call(
        _mlp_body,
        grid=(grid,),
        in_specs=[
            pl.BlockSpec((BLK, EMBED), rows),
            pl.BlockSpec((BLK, EMBED), rows),
            pl.BlockSpec((BLK, FEAT), rows),
            pl.BlockSpec((d_in, 64), full),
            pl.BlockSpec((1, 64), full),
            pl.BlockSpec((64, 32), full),
            pl.BlockSpec((1, 32), full),
            pl.BlockSpec((32, 1), full),
            pl.BlockSpec((1, 1), full),
            pl.BlockSpec((d_in, 1), full),
            pl.BlockSpec((1, 1), full),
        ],
        out_specs=pl.BlockSpec((BLK,), lambda i: (i,)),
        out_shape=jax.ShapeDtypeStruct((B,), jnp.float32),
    )(ue, ie, fe, W1, b1, W2, b2, W3, b3, Ww, bw)


def kernel(user_id, item_id, user_table, item_table, feat_table,
           W1, b1, W2, b2, W3, b3, Ww, bw):
    uid = user_id.astype(jnp.int32)
    iid = item_id.astype(jnp.int32)
    ue, ie, fe = _gather()(uid, iid, user_table, item_table, feat_table)
    return _mlp(ue, ie, fe,
                W1, b1.reshape(1, 64), W2, b2.reshape(1, 32),
                W3, b3.reshape(1, 1), Ww, bw.reshape(1, 1))


# speedup vs baseline: 1.6357x; 1.0358x over previous
import functools

import jax
import jax.numpy as jnp
from jax import lax
from jax.experimental import pallas as pl
from jax.experimental.pallas import tpu as pltpu
from jax.experimental.pallas import tpu_sc as plsc

B = 16384
EMBED = 64
FEAT = 16
N_USERS = 1000000
NC = 2
NS = 16
NW = NC * NS
BPW = B // NW
CH = 64
NCH = BPW // CH
T_MAX = N_USERS // 128 - 1
U_TAIL = (T_MAX + 1) * 128


def _gather_body(uid_hbm, iid_hbm, utT_hbm, it_hbm, ft_hbm,
                 ueT_hbm, ie_hbm, fe_hbm,
                 idx_u, idx_i, tbuf, out_u, buf_i, buf_f,
                 usem, sem, osem):
    c = lax.axis_index("c")
    s = lax.axis_index("s")
    wid = s * NC + c
    base = wid * BPW
    pltpu.sync_copy(uid_hbm.at[pl.ds(base, BPW)], idx_u.at[pl.ds(0, BPW)])
    pltpu.sync_copy(iid_hbm.at[pl.ds(base, BPW)], idx_i)
    idx_u[pl.ds(BPW, 16)] = jnp.full((16,), jnp.int32(0x7FFFFFF))

    def out_copies(k, p):
        ob = base + k * CH
        return (
            pltpu.make_async_copy(buf_i.at[p], ie_hbm.at[pl.ds(ob, CH)], osem),
            pltpu.make_async_copy(buf_f.at[p], fe_hbm.at[pl.ds(ob, CH)], osem),
        )

    def chunk(k, _):
        p = lax.rem(k, 2)
        descs = []
        for g in range(CH // 16):
            ivec = idx_i[pl.ds(k * CH + g * 16, 16)]
            for l in range(16):
                r = g * 16 + l
                i = ivec[l]
                descs.append(pltpu.async_copy(
                    it_hbm.at[pl.ds(i, 1), :], buf_i.at[p, pl.ds(r, 1), :],
                    sem))
                descs.append(pltpu.async_copy(
                    ft_hbm.at[pl.ds(i, 1), :], buf_f.at[p, pl.ds(r, 1), :],
                    sem))

        @pl.when(k >= 2)
        def _():
            for d in out_copies(k - 2, p):
                d.wait()
        for d in descs:
            d.wait()
        for d in out_copies(k, p):
            d.start()
        return ()

    lax.fori_loop(0, NCH, chunk, (), unroll=False)
    for k in (NCH - 2, NCH - 1):
        for d in out_copies(k, k % 2):
            d.wait()

    def tile_of(j):
        v = idx_u[pl.ds(j, 16)]
        return jnp.minimum(lax.shift_right_logical(v[0], 7), T_MAX)

    def tile_copy(t, p):
        off = pl.multiple_of(t * 128, 128)
        return pltpu.make_async_copy(
            utT_hbm.at[:, pl.ds(off, 128)], tbuf.at[p], usem)

    t0 = tile_of(0)
    t1 = tile_of(BPW - 16 + 15)
    tile_copy(t0, 0).start()
    tile_copy(t0, 0).wait()

    @pl.when(t0 < t1)
    def _():
        tile_copy(t0 + 1, 1).start()

    lanes = lax.iota(jnp.int32, 16)

    def step(_, state):
        t, j, p = state
        t_need = tile_of(j)
        adv = t < t_need

        @pl.when(adv)
        def _():
            tile_copy(t + 1, 1 - p).wait()

            @pl.when(t + 2 <= t1)
            def _():
                tile_copy(t + 2, p).start()

        @pl.when(jnp.logical_not(adv))
        def _():
            v = idx_u[pl.ds(j, 16)]
            lane = jnp.full((16,), lax.bitwise_and(v[0], 127))
            col = jnp.full((16,), j)
            for k in range(4):
                ev = lanes + (16 * k)
                vals = plsc.load_gather(tbuf.at[p], [ev, lane])
                plsc.store_scatter(out_u, [ev, col], vals)

        t2 = jnp.where(adv, t + 1, t)
        j2 = jnp.where(adv, j, j + 1)
        p2 = jnp.where(adv, 1 - p, p)
        return (t2, j2, p2)

    lax.fori_loop(0, (t1 - t0) + BPW, step,
                  (t0, jnp.int32(0), jnp.int32(0)), unroll=False)
    pltpu.sync_copy(out_u, ueT_hbm.at[:, pl.ds(base, BPW)])


@functools.cache
def _gather():
    return pl.kernel(
        _gather_body,
        out_type=(
            jax.ShapeDtypeStruct((EMBED, B), jnp.float32),
            jax.ShapeDtypeStruct((B, EMBED), jnp.float32),
            jax.ShapeDtypeStruct((B, FEAT), jnp.float32),
        ),
        mesh=plsc.VectorSubcoreMesh(
            core_axis_name="c", subcore_axis_name="s",
            num_cores=NC, num_subcores=NS),
        scratch_types=[
            pltpu.VMEM((BPW + 16,), jnp.int32),
            pltpu.VMEM((BPW,), jnp.int32),
            pltpu.VMEM((2, EMBED, 128), jnp.float32),
            pltpu.VMEM((EMBED, BPW), jnp.float32),
            pltpu.VMEM((2, CH, EMBED), jnp.float32),
            pltpu.VMEM((2, CH, FEAT), jnp.float32),
            pltpu.SemaphoreType.DMA,
            pltpu.SemaphoreType.DMA,
            pltpu.SemaphoreType.DMA,
        ],
        compiler_params=pltpu.CompilerParams(
            use_tc_tiling_on_sc=True, needs_layout_passes=False),
    )


BLK = 2048


def _mlp_body(uid, ueT, ie, fe, tail, W1, b1, W2, b2, W3, b3, Ww, bw, out):
    uid_r = uid[...]
    ueT_v = ueT[...]
    ie_v = ie[...]
    fe_v = fe[...]
    tail_off = uid_r - U_TAIL
    rows = lax.broadcasted_iota(jnp.int32, (EMBED, BLK), 0)
    ohT = (rows == tail_off).astype(jnp.float32)
    dg = lambda a, b: lax.dot_general(
        a, b, (((0,), (0,)), ((), ())),
        preferred_element_type=jnp.float32)
    corrT = dg(tail[...], ohT)
    ueT_u = jnp.where(uid_r >= U_TAIL, corrT, ueT_v)

    dot = functools.partial(jnp.dot, preferred_element_type=jnp.float32)
    h1 = dg(ueT_u, W1[:EMBED, :]) + dot(ie_v, W1[EMBED:2 * EMBED, :])
    h1 = h1 + dot(fe_v, W1[2 * EMBED:, :]) + b1[...]
    h1 = jnp.maximum(h1, 0.0)
    h2 = jnp.maximum(dot(h1, W2[...]) + b2[...], 0.0)
    deep = jnp.maximum(dot(h2, W3[...]) + b3[...], 0.0)
    wide = dg(ueT_u, Ww[:EMBED, :]) + dot(ie_v, Ww[EMBED:2 * EMBED, :])
    wide = wide + dot(fe_v, Ww[2 * EMBED:, :]) + bw[...]
    out[...] = (deep + wide)[:, 0]


def _mlp(uid_s, ueT, ie, fe, tail, W1, b1, W2, b2, W3, b3, Ww, bw):
    d_in = 2 * EMBED + FEAT
    grid = B // BLK
    rows = lambda i: (i, 0)
    cols = lambda i: (0, i)
    full = lambda i: (0, 0)
    return pl.pallas_call(
        _mlp_body,
        grid=(grid,),
        in_specs=[
            pl.BlockSpec((1, BLK), cols),
            pl.BlockSpec((EMBED, BLK), cols),
            pl.BlockSpec((BLK, EMBED), rows),
            pl.BlockSpec((BLK, FEAT), rows),
            pl.BlockSpec((EMBED, EMBED), full),
            pl.BlockSpec((d_in, 64), full),
            pl.BlockSpec((1, 64), full),
            pl.BlockSpec((64, 32), full),
            pl.BlockSpec((1, 32), full),
            pl.BlockSpec((32, 1), full),
            pl.BlockSpec((1, 1), full),
            pl.BlockSpec((d_in, 1), full),
            pl.BlockSpec((1, 1), full),
        ],
        out_specs=pl.BlockSpec((BLK,), lambda i: (i,)),
        out_shape=jax.ShapeDtypeStruct((B,), jnp.float32),
    )(uid_s, ueT, ie, fe, tail, W1, b1, W2, b2, W3, b3, Ww, bw)


def kernel(user_id, item_id, user_table, item_table, feat_table,
           W1, b1, W2, b2, W3, b3, Ww, bw):
    uid = user_id.astype(jnp.int32)
    iid = item_id.astype(jnp.int32)
    pos = lax.iota(jnp.int32, B)
    uid_s, iid_s, perm = lax.sort((uid, iid, pos), dimension=0, num_keys=1)
    tail = user_table[U_TAIL:, :]
    ueT_s, ie_s, fe_s = _gather()(uid_s, iid_s, user_table.T,
                                  item_table, feat_table)
    res_s = _mlp(uid_s.reshape(1, B), ueT_s, ie_s, fe_s, tail,
                 W1, b1.reshape(1, 64), W2, b2.reshape(1, 32),
                 W3, b3.reshape(1, 1), Ww, bw.reshape(1, 1))
    return jnp.zeros((B,), jnp.float32).at[perm].set(
        res_s, unique_indices=True, mode="promise_in_bounds")


# speedup vs baseline: 1.7949x; 1.0973x over previous
import functools

import jax
import jax.numpy as jnp
from jax import lax
from jax.experimental import pallas as pl
from jax.experimental.pallas import tpu as pltpu
from jax.experimental.pallas import tpu_sc as plsc

B = 16384
EMBED = 64
FEAT = 16
N_USERS = 1000000
NC = 2
NS = 16
NW = NC * NS
BPW = B // NW
CH = 64
NCH = BPW // CH
T_MAX = N_USERS // 128 - 1
U_TAIL = (T_MAX + 1) * 128
BT = 1
BW_ = BT * 128
B0_MAX = (N_USERS - BW_) // 128


def _scan_body(uid_hbm, utT_hbm, ueT_hbm, idx_u, bbuf, out_u, usem):
    c = lax.axis_index("c")
    s = lax.axis_index("s")
    wid = s * NC + c
    base = wid * BPW
    pltpu.sync_copy(uid_hbm.at[pl.ds(base, BPW)], idx_u.at[pl.ds(0, BPW)])
    idx_u[pl.ds(BPW, 16)] = jnp.full((16,), jnp.int32(0x7FFFFFF))

    def tile_of(j):
        v = idx_u[pl.ds(j, 16)]
        return jnp.minimum(lax.shift_right_logical(v[0], 7), T_MAX)

    t0 = tile_of(0)
    t1 = tile_of(BPW - 16 + 15)
    nb = lax.shift_right_logical(t1 - t0 + BT, BT.bit_length() - 1)

    def batch_copy(q, p):
        bt = jnp.minimum(t0 + q * BT, B0_MAX)
        off = pl.multiple_of(bt * 128, 128)
        return pltpu.make_async_copy(
            utT_hbm.at[:, pl.ds(off, BW_)], bbuf.at[p], usem)

    batch_copy(0, 0).start()
    batch_copy(0, 0).wait()

    @pl.when(nb > 1)
    def _():
        batch_copy(1, 1).start()

    lanes = lax.iota(jnp.int32, 16)

    def step(i, state):
        q, j = state
        b_lo = jnp.minimum(t0 + q * BT, B0_MAX) * 128
        b_hi = b_lo + BW_
        v = idx_u[pl.ds(j, 16)]
        last = q >= nb - 1
        in_hi = jnp.logical_or(v < b_hi, last)
        m = jnp.logical_and(
            jnp.logical_and(v >= b_lo, in_hi),
            lanes + j < BPW)
        cnt = plsc.all_reduce_population_count(m)[0]
        adv = jnp.logical_and(cnt == 0, jnp.logical_not(last))

        @pl.when(adv)
        def _():
            batch_copy(q + 1, lax.rem(q + 1, 2)).wait()

            @pl.when(q + 2 < nb)
            def _():
                batch_copy(q + 2, lax.rem(q, 2)).start()

        @pl.when(cnt > 0)
        def _():
            p = lax.rem(q, 2)
            col = jnp.clip(v - b_lo, 0, BW_ - 1)
            dst = lanes + j
            for e in range(EMBED):
                ev = jnp.full((16,), jnp.int32(e))
                vals = plsc.load_gather(bbuf.at[p], [ev, col])
                plsc.store_scatter(out_u, [ev, dst], vals)

        q2 = jnp.where(adv, q + 1, q)
        j2 = jnp.where(adv, j, j + cnt)
        return (q2, j2)

    lax.fori_loop(0, nb + BPW, step, (jnp.int32(0), jnp.int32(0)),
                  unroll=False)
    pltpu.sync_copy(out_u.at[:, pl.ds(0, BPW)],
                    ueT_hbm.at[:, pl.ds(base, BPW)])


@functools.cache
def _scan():
    return pl.kernel(
        _scan_body,
        out_type=jax.ShapeDtypeStruct((EMBED, B), jnp.float32),
        mesh=plsc.VectorSubcoreMesh(
            core_axis_name="c", subcore_axis_name="s",
            num_cores=NC, num_subcores=NS),
        scratch_types=[
            pltpu.VMEM((BPW + 16,), jnp.int32),
            pltpu.VMEM((2, EMBED, BW_), jnp.float32),
            pltpu.VMEM((EMBED, BPW + 128), jnp.float32),
            pltpu.SemaphoreType.DMA,
        ],
        compiler_params=pltpu.CompilerParams(
            use_tc_tiling_on_sc=True, needs_layout_passes=False),
    )


def _rows_body(iid_hbm, it_hbm, ft_hbm, ie_hbm, fe_hbm,
               idx_i, buf_i, buf_f, sem, osem):
    c = lax.axis_index("c")
    s = lax.axis_index("s")
    wid = s * NC + c
    base = wid * BPW
    pltpu.sync_copy(iid_hbm.at[pl.ds(base, BPW)], idx_i)

    def out_copies(k, p):
        ob = base + k * CH
        return (
            pltpu.make_async_copy(buf_i.at[p], ie_hbm.at[pl.ds(ob, CH)], osem),
            pltpu.make_async_copy(buf_f.at[p], fe_hbm.at[pl.ds(ob, CH)], osem),
        )

    def chunk(k, _):
        p = lax.rem(k, 2)
        descs = []
        for g in range(CH // 16):
            ivec = idx_i[pl.ds(k * CH + g * 16, 16)]
            for l in range(16):
                r = g * 16 + l
                i = ivec[l]
                descs.append(pltpu.async_copy(
                    it_hbm.at[pl.ds(i, 1), :], buf_i.at[p, pl.ds(r, 1), :],
                    sem))
                descs.append(pltpu.async_copy(
                    ft_hbm.at[pl.ds(i, 1), :], buf_f.at[p, pl.ds(r, 1), :],
                    sem))

        @pl.when(k >= 2)
        def _():
            for d in out_copies(k - 2, p):
                d.wait()
        for d in descs:
            d.wait()
        for d in out_copies(k, p):
            d.start()
        return ()

    lax.fori_loop(0, NCH, chunk, (), unroll=False)
    for k in (NCH - 2, NCH - 1):
        for d in out_copies(k, k % 2):
            d.wait()


@functools.cache
def _rows():
    return pl.kernel(
        _rows_body,
        out_type=(
            jax.ShapeDtypeStruct((B, EMBED), jnp.float32),
            jax.ShapeDtypeStruct((B, FEAT), jnp.float32),
        ),
        mesh=plsc.VectorSubcoreMesh(
            core_axis_name="c", subcore_axis_name="s",
            num_cores=NC, num_subcores=NS),
        scratch_types=[
            pltpu.VMEM((BPW,), jnp.int32),
            pltpu.VMEM((2, CH, EMBED), jnp.float32),
            pltpu.VMEM((2, CH, FEAT), jnp.float32),
            pltpu.SemaphoreType.DMA,
            pltpu.SemaphoreType.DMA,
        ],
        compiler_params=pltpu.CompilerParams(use_tc_tiling_on_sc=True),
    )


BLK = 2048


def _mlp_body(uid, ueT, ie, fe, tail, W1, b1, W2, b2, W3, b3, Ww, bw, out):
    uid_r = uid[...]
    ueT_v = ueT[...]
    ie_v = ie[...]
    fe_v = fe[...]
    tail_off = uid_r - U_TAIL
    rows = lax.broadcasted_iota(jnp.int32, (EMBED, BLK), 0)
    ohT = (rows == tail_off).astype(jnp.float32)
    dg = lambda a, b: lax.dot_general(
        a, b, (((0,), (0,)), ((), ())),
        preferred_element_type=jnp.float32)
    corrT = dg(tail[...], ohT)
    ueT_u = jnp.where(uid_r >= U_TAIL, corrT, ueT_v)

    dot = functools.partial(jnp.dot, preferred_element_type=jnp.float32)
    h1 = dg(ueT_u, W1[:EMBED, :]) + dot(ie_v, W1[EMBED:2 * EMBED, :])
    h1 = h1 + dot(fe_v, W1[2 * EMBED:, :]) + b1[...]
    h1 = jnp.maximum(h1, 0.0)
    h2 = jnp.maximum(dot(h1, W2[...]) + b2[...], 0.0)
    deep = jnp.maximum(dot(h2, W3[...]) + b3[...], 0.0)
    wide = dg(ueT_u, Ww[:EMBED, :]) + dot(ie_v, Ww[EMBED:2 * EMBED, :])
    wide = wide + dot(fe_v, Ww[2 * EMBED:, :]) + bw[...]
    out[...] = (deep + wide)[:, 0]


def _mlp(uid_s, ueT, ie, fe, tail, W1, b1, W2, b2, W3, b3, Ww, bw):
    d_in = 2 * EMBED + FEAT
    grid = B // BLK
    rows = lambda i: (i, 0)
    cols = lambda i: (0, i)
    full = lambda i: (0, 0)
    return pl.pallas_call(
        _mlp_body,
        grid=(grid,),
        in_specs=[
            pl.BlockSpec((1, BLK), cols),
            pl.BlockSpec((EMBED, BLK), cols),
            pl.BlockSpec((BLK, EMBED), rows),
            pl.BlockSpec((BLK, FEAT), rows),
            pl.BlockSpec((EMBED, EMBED), full),
            pl.BlockSpec((d_in, 64), full),
            pl.BlockSpec((1, 64), full),
            pl.BlockSpec((64, 32), full),
            pl.BlockSpec((1, 32), full),
            pl.BlockSpec((32, 1), full),
            pl.BlockSpec((1, 1), full),
            pl.BlockSpec((d_in, 1), full),
            pl.BlockSpec((1, 1), full),
        ],
        out_specs=pl.BlockSpec((BLK,), lambda i: (i,)),
        out_shape=jax.ShapeDtypeStruct((B,), jnp.float32),
    )(uid_s, ueT, ie, fe, tail, W1, b1, W2, b2, W3, b3, Ww, bw)


def kernel(user_id, item_id, user_table, item_table, feat_table,
           W1, b1, W2, b2, W3, b3, Ww, bw):
    uid = user_id.astype(jnp.int32)
    iid = item_id.astype(jnp.int32)
    pos = lax.iota(jnp.int32, B)
    uid_s, iid_s, perm = lax.sort((uid, iid, pos), dimension=0, num_keys=1)
    tail = user_table[U_TAIL:, :]
    ueT_s = _scan()(uid_s, user_table.T)
    ie_s, fe_s = _rows()(iid_s, item_table, feat_table)
    res_s = _mlp(uid_s.reshape(1, B), ueT_s, ie_s, fe_s, tail,
                 W1, b1.reshape(1, 64), W2, b2.reshape(1, 32),
                 W3, b3.reshape(1, 1), Ww, bw.reshape(1, 1))
    return lax.sort((perm, res_s), dimension=0, num_keys=1)[1]


# speedup vs baseline: 2.5390x; 1.4146x over previous
import functools

import jax
import jax.numpy as jnp
from jax import lax
from jax.experimental import pallas as pl
from jax.experimental.pallas import tpu as pltpu
from jax.experimental.pallas import tpu_sc as plsc

B = 16384
EMBED = 64
FEAT = 16
N_USERS = 1000000
NC = 2
NS = 16
NW = NC * NS
BPW = B // NW
CH = 64
NCH = BPW // CH
T_MAX = N_USERS // 128 - 1
U_TAIL = (T_MAX + 1) * 128
BT = 4
BW_ = BT * 128
B0_MAX = (N_USERS - BW_) // 128


def _scan_body(uid_hbm, utT_hbm, ueT_hbm, idx_u, bbuf, out_u, usem):
    c = lax.axis_index("c")
    s = lax.axis_index("s")
    wid = s * NC + c
    base = wid * BPW
    pltpu.sync_copy(uid_hbm.at[pl.ds(base, BPW)], idx_u.at[pl.ds(0, BPW)])
    idx_u[pl.ds(BPW, 16)] = jnp.full((16,), jnp.int32(0x7FFFFFF))

    def tile_of(j):
        v = idx_u[pl.ds(j, 16)]
        return jnp.minimum(lax.shift_right_logical(v[0], 7), T_MAX)

    t0 = tile_of(0)
    t1 = tile_of(BPW - 16 + 15)
    nb = lax.shift_right_logical(t1 - t0 + BT, BT.bit_length() - 1)

    def batch_copy(q, p):
        bt = jnp.minimum(t0 + q * BT, B0_MAX)
        off = pl.multiple_of(bt * 128, 128)
        return pltpu.make_async_copy(
            utT_hbm.at[:, pl.ds(off, BW_)], bbuf.at[p], usem)

    batch_copy(0, 0).start()
    batch_copy(0, 0).wait()

    @pl.when(nb > 1)
    def _():
        batch_copy(1, 1).start()

    lanes = lax.iota(jnp.int32, 16)

    def step(i, state):
        q, j = state
        b_lo = jnp.minimum(t0 + q * BT, B0_MAX) * 128
        b_hi = b_lo + BW_
        v = idx_u[pl.ds(j, 16)]
        last = q >= nb - 1
        in_hi = jnp.logical_or(v < b_hi, last)
        m = jnp.logical_and(
            jnp.logical_and(v >= b_lo, in_hi),
            lanes + j < BPW)
        cnt = plsc.all_reduce_population_count(m)[0]
        adv = jnp.logical_and(cnt == 0, jnp.logical_not(last))

        @pl.when(adv)
        def _():
            batch_copy(q + 1, lax.rem(q + 1, 2)).wait()

            @pl.when(q + 2 < nb)
            def _():
                batch_copy(q + 2, lax.rem(q, 2)).start()

        @pl.when(cnt > 0)
        def _():
            p = lax.rem(q, 2)
            col = jnp.clip(v - b_lo, 0, BW_ - 1)
            dst = lanes + j
            for e in range(EMBED):
                ev = jnp.full((16,), jnp.int32(e))
                vals = plsc.load_gather(bbuf.at[p], [ev, col])
                plsc.store_scatter(out_u, [ev, dst], vals)

        q2 = jnp.where(adv, q + 1, q)
        j2 = jnp.where(adv, j, j + cnt)
        return (q2, j2)

    lax.fori_loop(0, nb + BPW, step, (jnp.int32(0), jnp.int32(0)),
                  unroll=False)
    pltpu.sync_copy(out_u.at[:, pl.ds(0, BPW)],
                    ueT_hbm.at[:, pl.ds(base, BPW)])


@functools.cache
def _scan():
    return pl.kernel(
        _scan_body,
        out_type=jax.ShapeDtypeStruct((EMBED, B), jnp.float32),
        mesh=plsc.VectorSubcoreMesh(
            core_axis_name="c", subcore_axis_name="s",
            num_cores=NC, num_subcores=NS),
        scratch_types=[
            pltpu.VMEM((BPW + 16,), jnp.int32),
            pltpu.VMEM((2, EMBED, BW_), jnp.float32),
            pltpu.VMEM((EMBED, BPW + 128), jnp.float32),
            pltpu.SemaphoreType.DMA,
        ],
        compiler_params=pltpu.CompilerParams(
            use_tc_tiling_on_sc=True, needs_layout_passes=False),
    )


def _rows_body(iid_hbm, it_hbm, ft_hbm, ie_hbm, fe_hbm,
               idx_i, buf_i, buf_f, sem, osem):
    c = lax.axis_index("c")
    s = lax.axis_index("s")
    wid = s * NC + c
    base = wid * BPW
    pltpu.sync_copy(iid_hbm.at[pl.ds(base, BPW)], idx_i)

    def out_copies(k, p):
        ob = base + k * CH
        return (
            pltpu.make_async_copy(buf_i.at[p], ie_hbm.at[pl.ds(ob, CH)], osem),
            pltpu.make_async_copy(buf_f.at[p], fe_hbm.at[pl.ds(ob, CH)], osem),
        )

    def chunk(k, _):
        p = lax.rem(k, 2)
        descs = []
        for g in range(CH // 16):
            ivec = idx_i[pl.ds(k * CH + g * 16, 16)]
            for l in range(16):
                r = g * 16 + l
                i = ivec[l]
                descs.append(pltpu.async_copy(
                    it_hbm.at[pl.ds(i, 1), :], buf_i.at[p, pl.ds(r, 1), :],
                    sem))
                descs.append(pltpu.async_copy(
                    ft_hbm.at[pl.ds(i, 1), :], buf_f.at[p, pl.ds(r, 1), :],
                    sem))

        @pl.when(k >= 2)
        def _():
            for d in out_copies(k - 2, p):
                d.wait()
        for d in descs:
            d.wait()
        for d in out_copies(k, p):
            d.start()
        return ()

    lax.fori_loop(0, NCH, chunk, (), unroll=False)
    for k in (NCH - 2, NCH - 1):
        for d in out_copies(k, k % 2):
            d.wait()


@functools.cache
def _rows():
    return pl.kernel(
        _rows_body,
        out_type=(
            jax.ShapeDtypeStruct((B, EMBED), jnp.float32),
            jax.ShapeDtypeStruct((B, FEAT), jnp.float32),
        ),
        mesh=plsc.VectorSubcoreMesh(
            core_axis_name="c", subcore_axis_name="s",
            num_cores=NC, num_subcores=NS),
        scratch_types=[
            pltpu.VMEM((BPW,), jnp.int32),
            pltpu.VMEM((2, CH, EMBED), jnp.float32),
            pltpu.VMEM((2, CH, FEAT), jnp.float32),
            pltpu.SemaphoreType.DMA,
            pltpu.SemaphoreType.DMA,
        ],
        compiler_params=pltpu.CompilerParams(use_tc_tiling_on_sc=True),
    )


BLK = 2048


def _mlp_body(uid, ueT, ie, fe, tail, W1, b1, W2, b2, W3, b3, Ww, bw, out):
    uid_r = uid[...]
    ueT_v = ueT[...]
    ie_v = ie[...]
    fe_v = fe[...]
    tail_off = uid_r - U_TAIL
    rows = lax.broadcasted_iota(jnp.int32, (EMBED, BLK), 0)
    ohT = (rows == tail_off).astype(jnp.float32)
    dg = lambda a, b: lax.dot_general(
        a, b, (((0,), (0,)), ((), ())),
        preferred_element_type=jnp.float32)
    corrT = dg(tail[...], ohT)
    ueT_u = jnp.where(uid_r >= U_TAIL, corrT, ueT_v)

    dot = functools.partial(jnp.dot, preferred_element_type=jnp.float32)
    h1 = dg(ueT_u, W1[:EMBED, :]) + dot(ie_v, W1[EMBED:2 * EMBED, :])
    h1 = h1 + dot(fe_v, W1[2 * EMBED:, :]) + b1[...]
    h1 = jnp.maximum(h1, 0.0)
    h2 = jnp.maximum(dot(h1, W2[...]) + b2[...], 0.0)
    deep = jnp.maximum(dot(h2, W3[...]) + b3[...], 0.0)
    wide = dg(ueT_u, Ww[:EMBED, :]) + dot(ie_v, Ww[EMBED:2 * EMBED, :])
    wide = wide + dot(fe_v, Ww[2 * EMBED:, :]) + bw[...]
    out[...] = (deep + wide)[:, 0]


def _mlp(uid_s, ueT, ie, fe, tail, W1, b1, W2, b2, W3, b3, Ww, bw):
    d_in = 2 * EMBED + FEAT
    grid = B // BLK
    rows = lambda i: (i, 0)
    cols = lambda i: (0, i)
    full = lambda i: (0, 0)
    return pl.pallas_call(
        _mlp_body,
        grid=(grid,),
        in_specs=[
            pl.BlockSpec((1, BLK), cols),
            pl.BlockSpec((EMBED, BLK), cols),
            pl.BlockSpec((BLK, EMBED), rows),
            pl.BlockSpec((BLK, FEAT), rows),
            pl.BlockSpec((EMBED, EMBED), full),
            pl.BlockSpec((d_in, 64), full),
            pl.BlockSpec((1, 64), full),
            pl.BlockSpec((64, 32), full),
            pl.BlockSpec((1, 32), full),
            pl.BlockSpec((32, 1), full),
            pl.BlockSpec((1, 1), full),
            pl.BlockSpec((d_in, 1), full),
            pl.BlockSpec((1, 1), full),
        ],
        out_specs=pl.BlockSpec((BLK,), lambda i: (i,)),
        out_shape=jax.ShapeDtypeStruct((B,), jnp.float32),
    )(uid_s, ueT, ie, fe, tail, W1, b1, W2, b2, W3, b3, Ww, bw)


def kernel(user_id, item_id, user_table, item_table, feat_table,
           W1, b1, W2, b2, W3, b3, Ww, bw):
    uid = user_id.astype(jnp.int32)
    iid = item_id.astype(jnp.int32)
    pos = lax.iota(jnp.int32, B)
    uid_s, iid_s, perm = lax.sort((uid, iid, pos), dimension=0, num_keys=1)
    tail = user_table[U_TAIL:, :]
    ueT_s = _scan()(uid_s, user_table.T)
    ie_s, fe_s = _rows()(iid_s, item_table, feat_table)
    res_s = _mlp(uid_s.reshape(1, B), ueT_s, ie_s, fe_s, tail,
                 W1, b1.reshape(1, 64), W2, b2.reshape(1, 32),
                 W3, b3.reshape(1, 1), Ww, bw.reshape(1, 1))
    return lax.sort((perm, res_s), dimension=0, num_keys=1)[1]


# speedup vs baseline: 2.9494x; 1.1616x over previous
import functools

import jax
import jax.numpy as jnp
from jax import lax
from jax.experimental import pallas as pl
from jax.experimental.pallas import tpu as pltpu
from jax.experimental.pallas import tpu_sc as plsc

B = 16384
EMBED = 64
FEAT = 16
N_USERS = 1000000
NC = 2
NS = 16
NW = NC * NS
BPW = B // NW
CH = 64
NCH = BPW // CH
T_MAX = N_USERS // 128 - 1
U_TAIL = (T_MAX + 1) * 128
BT = 4
BW_ = BT * 128
B0_MAX = (N_USERS - BW_) // 128


def _scan_body(uid_hbm, utT_hbm, ueT_hbm, idx_u, bbuf, out_u, usem):
    c = lax.axis_index("c")
    s = lax.axis_index("s")
    wid = s * NC + c
    base = wid * BPW
    pltpu.sync_copy(uid_hbm.at[pl.ds(base, BPW)], idx_u.at[pl.ds(0, BPW)])
    idx_u[pl.ds(BPW, 16)] = jnp.full((16,), jnp.int32(0x7FFFFFF))

    def tile_of(j):
        v = idx_u[pl.ds(j, 16)]
        return jnp.minimum(lax.shift_right_logical(v[0], 7), T_MAX)

    t0 = tile_of(0)
    t1 = tile_of(BPW - 16 + 15)
    nb = lax.shift_right_logical(t1 - t0 + BT, BT.bit_length() - 1)

    def batch_copy(q, p):
        bt = jnp.minimum(t0 + q * BT, B0_MAX)
        off = pl.multiple_of(bt * 128, 128)
        return pltpu.make_async_copy(
            utT_hbm.at[:, pl.ds(off, BW_)], bbuf.at[p], usem)

    batch_copy(0, 0).start()
    batch_copy(0, 0).wait()

    @pl.when(nb > 1)
    def _():
        batch_copy(1, 1).start()

    lanes = lax.iota(jnp.int32, 16)

    def step(i, state):
        q, j = state
        b_lo = jnp.minimum(t0 + q * BT, B0_MAX) * 128
        b_hi = b_lo + BW_
        v = idx_u[pl.ds(j, 16)]
        last = q >= nb - 1
        in_hi = jnp.logical_or(v < b_hi, last)
        m = jnp.logical_and(
            jnp.logical_and(v >= b_lo, in_hi),
            lanes + j < BPW)
        cnt = plsc.all_reduce_population_count(m)[0]
        adv = jnp.logical_and(cnt == 0, jnp.logical_not(last))

        @pl.when(adv)
        def _():
            @pl.when(q + 2 < nb)
            def _():
                batch_copy(q + 2, lax.rem(q, 2)).start()

            batch_copy(q + 1, lax.rem(q + 1, 2)).wait()

        @pl.when(cnt > 0)
        def _():
            p = lax.rem(q, 2)
            col = jnp.clip(v - b_lo, 0, BW_ - 1)
            dst = lanes + j
            for e in range(EMBED):
                ev = jnp.full((16,), jnp.int32(e))
                vals = plsc.load_gather(bbuf.at[p], [ev, col])
                plsc.store_scatter(out_u, [ev, dst], vals)

        q2 = jnp.where(adv, q + 1, q)
        j2 = jnp.where(adv, j, j + cnt)
        return (q2, j2)

    lax.fori_loop(0, nb + BPW, step, (jnp.int32(0), jnp.int32(0)),
                  unroll=False)
    pltpu.sync_copy(out_u.at[:, pl.ds(0, BPW)],
                    ueT_hbm.at[:, pl.ds(base, BPW)])


@functools.cache
def _scan():
    return pl.kernel(
        _scan_body,
        out_type=jax.ShapeDtypeStruct((EMBED, B), jnp.float32),
        mesh=plsc.VectorSubcoreMesh(
            core_axis_name="c", subcore_axis_name="s",
            num_cores=NC, num_subcores=NS),
        scratch_types=[
            pltpu.VMEM((BPW + 16,), jnp.int32),
            pltpu.VMEM((2, EMBED, BW_), jnp.float32),
            pltpu.VMEM((EMBED, BPW + 128), jnp.float32),
            pltpu.SemaphoreType.DMA,
        ],
        compiler_params=pltpu.CompilerParams(
            use_tc_tiling_on_sc=True, needs_layout_passes=False),
    )


def _rows_body(iid_hbm, it_hbm, ft_hbm, ie_hbm, fe_hbm,
               idx_i, buf_i, buf_f, sem, osem):
    c = lax.axis_index("c")
    s = lax.axis_index("s")
    wid = s * NC + c
    base = wid * BPW
    pltpu.sync_copy(iid_hbm.at[pl.ds(base, BPW)], idx_i)

    def out_copies(k, p):
        ob = base + k * CH
        return (
            pltpu.make_async_copy(buf_i.at[p], ie_hbm.at[pl.ds(ob, CH)], osem),
            pltpu.make_async_copy(buf_f.at[p], fe_hbm.at[pl.ds(ob, CH)], osem),
        )

    def chunk(k, _):
        p = lax.rem(k, 2)
        descs = []
        for g in range(CH // 16):
            ivec = idx_i[pl.ds(k * CH + g * 16, 16)]
            for l in range(16):
                r = g * 16 + l
                i = ivec[l]
                descs.append(pltpu.async_copy(
                    it_hbm.at[pl.ds(i, 1), :], buf_i.at[p, pl.ds(r, 1), :],
                    sem))
                descs.append(pltpu.async_copy(
                    ft_hbm.at[pl.ds(i, 1), :], buf_f.at[p, pl.ds(r, 1), :],
                    sem))

        @pl.when(k >= 2)
        def _():
            for d in out_copies(k - 2, p):
                d.wait()
        for d in descs:
            d.wait()
        for d in out_copies(k, p):
            d.start()
        return ()

    lax.fori_loop(0, NCH, chunk, (), unroll=False)
    for k in (NCH - 2, NCH - 1):
        for d in out_copies(k, k % 2):
            d.wait()


@functools.cache
def _rows():
    return pl.kernel(
        _rows_body,
        out_type=(
            jax.ShapeDtypeStruct((B, EMBED), jnp.float32),
            jax.ShapeDtypeStruct((B, FEAT), jnp.float32),
        ),
        mesh=plsc.VectorSubcoreMesh(
            core_axis_name="c", subcore_axis_name="s",
            num_cores=NC, num_subcores=NS),
        scratch_types=[
            pltpu.VMEM((BPW,), jnp.int32),
            pltpu.VMEM((2, CH, EMBED), jnp.float32),
            pltpu.VMEM((2, CH, FEAT), jnp.float32),
            pltpu.SemaphoreType.DMA,
            pltpu.SemaphoreType.DMA,
        ],
        compiler_params=pltpu.CompilerParams(use_tc_tiling_on_sc=True),
    )


BLK = 2048


def _mlp_body(uid, ueT, ie, fe, tail, W1, b1, W2, b2, W3, b3, Ww, bw, out):
    uid_r = uid[...]
    ueT_v = ueT[...]
    ie_v = ie[...]
    fe_v = fe[...]
    tail_off = uid_r - U_TAIL
    rows = lax.broadcasted_iota(jnp.int32, (EMBED, BLK), 0)
    ohT = (rows == tail_off).astype(jnp.float32)
    dg = lambda a, b: lax.dot_general(
        a, b, (((0,), (0,)), ((), ())),
        preferred_element_type=jnp.float32)
    corrT = dg(tail[...], ohT)
    ueT_u = jnp.where(uid_r >= U_TAIL, corrT, ueT_v)

    dot = functools.partial(jnp.dot, preferred_element_type=jnp.float32)
    h1 = dg(ueT_u, W1[:EMBED, :]) + dot(ie_v, W1[EMBED:2 * EMBED, :])
    h1 = h1 + dot(fe_v, W1[2 * EMBED:, :]) + b1[...]
    h1 = jnp.maximum(h1, 0.0)
    h2 = jnp.maximum(dot(h1, W2[...]) + b2[...], 0.0)
    deep = jnp.maximum(dot(h2, W3[...]) + b3[...], 0.0)
    wide = dg(ueT_u, Ww[:EMBED, :]) + dot(ie_v, Ww[EMBED:2 * EMBED, :])
    wide = wide + dot(fe_v, Ww[2 * EMBED:, :]) + bw[...]
    out[...] = (deep + wide)[:, 0]


def _mlp(uid_s, ueT, ie, fe, tail, W1, b1, W2, b2, W3, b3, Ww, bw):
    d_in = 2 * EMBED + FEAT
    grid = B // BLK
    rows = lambda i: (i, 0)
    cols = lambda i: (0, i)
    full = lambda i: (0, 0)
    return pl.pallas_call(
        _mlp_body,
        grid=(grid,),
        in_specs=[
            pl.BlockSpec((1, BLK), cols),
            pl.BlockSpec((EMBED, BLK), cols),
            pl.BlockSpec((BLK, EMBED), rows),
            pl.BlockSpec((BLK, FEAT), rows),
            pl.BlockSpec((EMBED, EMBED), full),
            pl.BlockSpec((d_in, 64), full),
            pl.BlockSpec((1, 64), full),
            pl.BlockSpec((64, 32), full),
            pl.BlockSpec((1, 32), full),
            pl.BlockSpec((32, 1), full),
            pl.BlockSpec((1, 1), full),
            pl.BlockSpec((d_in, 1), full),
            pl.BlockSpec((1, 1), full),
        ],
        out_specs=pl.BlockSpec((BLK,), lambda i: (i,)),
        out_shape=jax.ShapeDtypeStruct((B,), jnp.float32),
    )(uid_s, ueT, ie, fe, tail, W1, b1, W2, b2, W3, b3, Ww, bw)


def kernel(user_id, item_id, user_table, item_table, feat_table,
           W1, b1, W2, b2, W3, b3, Ww, bw):
    uid = user_id.astype(jnp.int32)
    iid = item_id.astype(jnp.int32)
    pos = lax.iota(jnp.int32, B)
    uid_s, iid_s, perm = lax.sort((uid, iid, pos), dimension=0, num_keys=1)
    tail = user_table[U_TAIL:, :]
    ueT_s = _scan()(uid_s, user_table.T)
    ie_s, fe_s = _rows()(iid_s, item_table, feat_table)
    res_s = _mlp(uid_s.reshape(1, B), ueT_s, ie_s, fe_s, tail,
                 W1, b1.reshape(1, 64), W2, b2.reshape(1, 32),
                 W3, b3.reshape(1, 1), Ww, bw.reshape(1, 1))
    return lax.sort((perm, res_s), dimension=0, num_keys=1)[1]


# speedup vs baseline: 2.9772x; 1.0094x over previous
import functools

import jax
import jax.numpy as jnp
from jax import lax
from jax.experimental import pallas as pl
from jax.experimental.pallas import tpu as pltpu
from jax.experimental.pallas import tpu_sc as plsc

B = 16384
EMBED = 64
FEAT = 16
N_USERS = 1000000
NC = 2
NS = 16
NW = NC * NS
BPW = B // NW
CH = 64
NCH = BPW // CH
T_MAX = N_USERS // 128 - 1
U_TAIL = (T_MAX + 1) * 128
BT = 2
BW_ = BT * 128
B0_MAX = (N_USERS - BW_) // 128


def _scan_body(uid_hbm, utT_hbm, ueT_hbm, idx_u, bbuf, out_u, usem):
    c = lax.axis_index("c")
    s = lax.axis_index("s")
    wid = s * NC + c
    base = wid * BPW
    pltpu.sync_copy(uid_hbm.at[pl.ds(base, BPW)], idx_u.at[pl.ds(0, BPW)])
    idx_u[pl.ds(BPW, 16)] = jnp.full((16,), jnp.int32(0x7FFFFFF))

    def tile_of(j):
        v = idx_u[pl.ds(j, 16)]
        return jnp.minimum(lax.shift_right_logical(v[0], 7), T_MAX)

    t0 = tile_of(0)
    t1 = tile_of(BPW - 16 + 15)
    nb = lax.shift_right_logical(t1 - t0 + BT, BT.bit_length() - 1)

    def batch_copy(q, p):
        bt = jnp.minimum(t0 + q * BT, B0_MAX)
        off = pl.multiple_of(bt * 128, 128)
        return pltpu.make_async_copy(
            utT_hbm.at[:, pl.ds(off, BW_)], bbuf.at[p], usem)

    batch_copy(0, 0).start()

    @pl.when(nb > 1)
    def _():
        batch_copy(1, 1).start()

    @pl.when(nb > 2)
    def _():
        batch_copy(2, 2).start()

    batch_copy(0, 0).wait()

    lanes = lax.iota(jnp.int32, 16)

    def step(i, state):
        q, j = state
        b_lo = jnp.minimum(t0 + q * BT, B0_MAX) * 128
        b_hi = b_lo + BW_
        v = idx_u[pl.ds(j, 16)]
        last = q >= nb - 1
        in_hi = jnp.logical_or(v < b_hi, last)
        m = jnp.logical_and(
            jnp.logical_and(v >= b_lo, in_hi),
            lanes + j < BPW)
        cnt = plsc.all_reduce_population_count(m)[0]
        adv = jnp.logical_and(cnt == 0, jnp.logical_not(last))

        @pl.when(adv)
        def _():
            @pl.when(q + 3 < nb)
            def _():
                batch_copy(q + 3, lax.rem(q, 3)).start()

            batch_copy(q + 1, lax.rem(q + 1, 3)).wait()

        @pl.when(cnt > 0)
        def _():
            p = lax.rem(q, 3)
            col = jnp.clip(v - b_lo, 0, BW_ - 1)
            dst = lanes + j
            for e in range(EMBED):
                ev = jnp.full((16,), jnp.int32(e))
                vals = plsc.load_gather(bbuf.at[p], [ev, col])
                plsc.store_scatter(out_u, [ev, dst], vals)

        q2 = jnp.where(adv, q + 1, q)
        j2 = jnp.where(adv, j, j + cnt)
        return (q2, j2)

    lax.fori_loop(0, nb + BPW, step, (jnp.int32(0), jnp.int32(0)),
                  unroll=False)
    pltpu.sync_copy(out_u.at[:, pl.ds(0, BPW)],
                    ueT_hbm.at[:, pl.ds(base, BPW)])


@functools.cache
def _scan():
    return pl.kernel(
        _scan_body,
        out_type=jax.ShapeDtypeStruct((EMBED, B), jnp.float32),
        mesh=plsc.VectorSubcoreMesh(
            core_axis_name="c", subcore_axis_name="s",
            num_cores=NC, num_subcores=NS),
        scratch_types=[
            pltpu.VMEM((BPW + 16,), jnp.int32),
            pltpu.VMEM((3, EMBED, BW_), jnp.float32),
            pltpu.VMEM((EMBED, BPW + 128), jnp.float32),
            pltpu.SemaphoreType.DMA,
        ],
        compiler_params=pltpu.CompilerParams(
            use_tc_tiling_on_sc=True, needs_layout_passes=False),
    )


def _rows_body(iid_hbm, it_hbm, ft_hbm, ie_hbm, fe_hbm,
               idx_i, buf_i, buf_f, sem, osem):
    c = lax.axis_index("c")
    s = lax.axis_index("s")
    wid = s * NC + c
    base = wid * BPW
    pltpu.sync_copy(iid_hbm.at[pl.ds(base, BPW)], idx_i)

    def out_copies(k, p):
        ob = base + k * CH
        return (
            pltpu.make_async_copy(buf_i.at[p], ie_hbm.at[pl.ds(ob, CH)], osem),
            pltpu.make_async_copy(buf_f.at[p], fe_hbm.at[pl.ds(ob, CH)], osem),
        )

    def chunk(k, _):
        p = lax.rem(k, 2)
        descs = []
        for g in range(CH // 16):
            ivec = idx_i[pl.ds(k * CH + g * 16, 16)]
            for l in range(16):
                r = g * 16 + l
                i = ivec[l]
                descs.append(pltpu.async_copy(
                    it_hbm.at[pl.ds(i, 1), :], buf_i.at[p, pl.ds(r, 1), :],
                    sem))
                descs.append(pltpu.async_copy(
                    ft_hbm.at[pl.ds(i, 1), :], buf_f.at[p, pl.ds(r, 1), :],
                    sem))

        @pl.when(k >= 2)
        def _():
            for d in out_copies(k - 2, p):
                d.wait()
        for d in descs:
            d.wait()
        for d in out_copies(k, p):
            d.start()
        return ()

    lax.fori_loop(0, NCH, chunk, (), unroll=False)
    for k in (NCH - 2, NCH - 1):
        for d in out_copies(k, k % 2):
            d.wait()


@functools.cache
def _rows():
    return pl.kernel(
        _rows_body,
        out_type=(
            jax.ShapeDtypeStruct((B, EMBED), jnp.float32),
            jax.ShapeDtypeStruct((B, FEAT), jnp.float32),
        ),
        mesh=plsc.VectorSubcoreMesh(
            core_axis_name="c", subcore_axis_name="s",
            num_cores=NC, num_subcores=NS),
        scratch_types=[
            pltpu.VMEM((BPW,), jnp.int32),
            pltpu.VMEM((2, CH, EMBED), jnp.float32),
            pltpu.VMEM((2, CH, FEAT), jnp.float32),
            pltpu.SemaphoreType.DMA,
            pltpu.SemaphoreType.DMA,
        ],
        compiler_params=pltpu.CompilerParams(use_tc_tiling_on_sc=True),
    )


BLK = 2048


def _mlp_body(uid, ueT, ie, fe, tail, W1, b1, W2, b2, W3, b3, Ww, bw, out):
    uid_r = uid[...]
    ueT_v = ueT[...]
    ie_v = ie[...]
    fe_v = fe[...]
    tail_off = uid_r - U_TAIL
    rows = lax.broadcasted_iota(jnp.int32, (EMBED, BLK), 0)
    ohT = (rows == tail_off).astype(jnp.float32)
    dg = lambda a, b: lax.dot_general(
        a, b, (((0,), (0,)), ((), ())),
        preferred_element_type=jnp.float32)
    corrT = dg(tail[...], ohT)
    ueT_u = jnp.where(uid_r >= U_TAIL, corrT, ueT_v)

    dot = functools.partial(jnp.dot, preferred_element_type=jnp.float32)
    h1 = dg(ueT_u, W1[:EMBED, :]) + dot(ie_v, W1[EMBED:2 * EMBED, :])
    h1 = h1 + dot(fe_v, W1[2 * EMBED:, :]) + b1[...]
    h1 = jnp.maximum(h1, 0.0)
    h2 = jnp.maximum(dot(h1, W2[...]) + b2[...], 0.0)
    deep = jnp.maximum(dot(h2, W3[...]) + b3[...], 0.0)
    wide = dg(ueT_u, Ww[:EMBED, :]) + dot(ie_v, Ww[EMBED:2 * EMBED, :])
    wide = wide + dot(fe_v, Ww[2 * EMBED:, :]) + bw[...]
    out[...] = (deep + wide)[:, 0]


def _mlp(uid_s, ueT, ie, fe, tail, W1, b1, W2, b2, W3, b3, Ww, bw):
    d_in = 2 * EMBED + FEAT
    grid = B // BLK
    rows = lambda i: (i, 0)
    cols = lambda i: (0, i)
    full = lambda i: (0, 0)
    return pl.pallas_call(
        _mlp_body,
        grid=(grid,),
        in_specs=[
            pl.BlockSpec((1, BLK), cols),
            pl.BlockSpec((EMBED, BLK), cols),
            pl.BlockSpec((BLK, EMBED), rows),
            pl.BlockSpec((BLK, FEAT), rows),
            pl.BlockSpec((EMBED, EMBED), full),
            pl.BlockSpec((d_in, 64), full),
            pl.BlockSpec((1, 64), full),
            pl.BlockSpec((64, 32), full),
            pl.BlockSpec((1, 32), full),
            pl.BlockSpec((32, 1), full),
            pl.BlockSpec((1, 1), full),
            pl.BlockSpec((d_in, 1), full),
            pl.BlockSpec((1, 1), full),
        ],
        out_specs=pl.BlockSpec((BLK,), lambda i: (i,)),
        out_shape=jax.ShapeDtypeStruct((B,), jnp.float32),
    )(uid_s, ueT, ie, fe, tail, W1, b1, W2, b2, W3, b3, Ww, bw)


def kernel(user_id, item_id, user_table, item_table, feat_table,
           W1, b1, W2, b2, W3, b3, Ww, bw):
    uid = user_id.astype(jnp.int32)
    iid = item_id.astype(jnp.int32)
    pos = lax.iota(jnp.int32, B)
    uid_s, iid_s, perm = lax.sort((uid, iid, pos), dimension=0, num_keys=1)
    tail = user_table[U_TAIL:, :]
    ueT_s = _scan()(uid_s, user_table.T)
    ie_s, fe_s = _rows()(iid_s, item_table, feat_table)
    res_s = _mlp(uid_s.reshape(1, B), ueT_s, ie_s, fe_s, tail,
                 W1, b1.reshape(1, 64), W2, b2.reshape(1, 32),
                 W3, b3.reshape(1, 1), Ww, bw.reshape(1, 1))
    return lax.sort((perm, res_s), dimension=0, num_keys=1)[1]


# speedup vs baseline: 3.0739x; 1.0325x over previous
import functools

import jax
import jax.numpy as jnp
from jax import lax
from jax.experimental import pallas as pl
from jax.experimental.pallas import tpu as pltpu
from jax.experimental.pallas import tpu_sc as plsc

B = 16384
EMBED = 64
FEAT = 16
N_USERS = 1000000
NC = 2
NS = 16
NW = NC * NS
BPW = B // NW
CH = 64
NCH = BPW // CH
T_MAX = N_USERS // 128 - 1
U_TAIL = (T_MAX + 1) * 128
BT = 2
BW_ = BT * 128
B0_MAX = (N_USERS - BW_) // 128


def _scan_body(uid_hbm, utT_hbm, ueT_hbm, idx_u, bbuf, out_u, usem):
    c = lax.axis_index("c")
    s = lax.axis_index("s")
    wid = s * NC + c
    base = wid * BPW
    pltpu.sync_copy(uid_hbm.at[pl.ds(base, BPW)], idx_u.at[pl.ds(0, BPW)])
    idx_u[pl.ds(BPW, 16)] = jnp.full((16,), jnp.int32(0x7FFFFFF))

    def tile_of(j):
        v = idx_u[pl.ds(j, 16)]
        return jnp.minimum(lax.shift_right_logical(v[0], 7), T_MAX)

    t0 = tile_of(0)
    t1 = tile_of(BPW - 16 + 15)
    nb = lax.shift_right_logical(t1 - t0 + BT, BT.bit_length() - 1)

    def batch_copy(q, p):
        bt = jnp.minimum(t0 + q * BT, B0_MAX)
        off = pl.multiple_of(bt * 128, 128)
        return pltpu.make_async_copy(
            utT_hbm.at[:, pl.ds(off, BW_)], bbuf.at[p], usem)

    batch_copy(0, 0).start()

    @pl.when(nb > 1)
    def _():
        batch_copy(1, 1).start()

    @pl.when(nb > 2)
    def _():
        batch_copy(2, 2).start()

    batch_copy(0, 0).wait()

    lanes = lax.iota(jnp.int32, 16)

    def step(i, state):
        q, j = state
        b_lo = jnp.minimum(t0 + q * BT, B0_MAX) * 128
        b_hi = b_lo + BW_
        v = idx_u[pl.ds(j, 16)]
        last = q >= nb - 1
        in_hi = jnp.logical_or(v < b_hi, last)
        m = jnp.logical_and(
            jnp.logical_and(v >= b_lo, in_hi),
            lanes + j < BPW)
        cnt = plsc.all_reduce_population_count(m)[0]
        adv = jnp.logical_and(cnt == 0, jnp.logical_not(last))

        @pl.when(adv)
        def _():
            @pl.when(q + 3 < nb)
            def _():
                batch_copy(q + 3, lax.rem(q, 3)).start()

            batch_copy(q + 1, lax.rem(q + 1, 3)).wait()

        @pl.when(cnt > 0)
        def _():
            p = lax.rem(q, 3)
            col = jnp.clip(v - b_lo, 0, BW_ - 1)
            dst = lanes + j
            for e in range(EMBED):
                ev = jnp.full((16,), jnp.int32(e))
                vals = plsc.load_gather(bbuf.at[p], [ev, col])
                plsc.store_scatter(out_u, [ev, dst], vals)

        q2 = jnp.where(adv, q + 1, q)
        j2 = jnp.where(adv, j, j + cnt)
        return (q2, j2)

    lax.fori_loop(0, nb + BPW, step, (jnp.int32(0), jnp.int32(0)),
                  unroll=False)
    pltpu.sync_copy(out_u.at[:, pl.ds(0, BPW)],
                    ueT_hbm.at[:, pl.ds(base, BPW)])


@functools.cache
def _scan():
    return pl.kernel(
        _scan_body,
        out_type=jax.ShapeDtypeStruct((EMBED, B), jnp.float32),
        mesh=plsc.VectorSubcoreMesh(
            core_axis_name="c", subcore_axis_name="s",
            num_cores=NC, num_subcores=NS),
        scratch_types=[
            pltpu.VMEM((BPW + 16,), jnp.int32),
            pltpu.VMEM((3, EMBED, BW_), jnp.float32),
            pltpu.VMEM((EMBED, BPW + 128), jnp.float32),
            pltpu.SemaphoreType.DMA,
        ],
        compiler_params=pltpu.CompilerParams(
            use_tc_tiling_on_sc=True, needs_layout_passes=False),
    )


def _rows_body(iid_hbm, it_hbm, fp_hbm, ieT_hbm, feT_hbm,
               idx_i, buf_i, buf_f, obuf_i, obuf_f, sem):
    c = lax.axis_index("c")
    s = lax.axis_index("s")
    wid = s * NC + c
    base = wid * BPW
    pltpu.sync_copy(iid_hbm.at[pl.ds(base, BPW)], idx_i)
    lanes = lax.iota(jnp.int32, 16)

    def chunk(k, _):
        p = lax.rem(k, 2)
        descs = []
        for g in range(CH // 16):
            ivec = idx_i[pl.ds(k * CH + g * 16, 16)]
            for l in range(16):
                r = g * 16 + l
                i = ivec[l]
                descs.append(pltpu.async_copy(
                    it_hbm.at[pl.ds(i, 1), :],
                    buf_i.at[p, pl.ds(r, 1), :], sem))
                descs.append(pltpu.async_copy(
                    fp_hbm.at[pl.ds(lax.shift_right_logical(i, 3), 1), :],
                    buf_f.at[p, pl.ds(r, 1), :], sem))
        for d in descs:
            d.wait()
        for g in range(CH // 16):
            ivec = idx_i[pl.ds(k * CH + g * 16, 16)]
            for l in range(16):
                r = g * 16 + l
                col = jnp.full((16,), k * CH + r)
                for e4 in range(EMBED // 16):
                    vals = buf_i[p, r, pl.ds(16 * e4, 16)]
                    plsc.store_scatter(obuf_i, [lanes + 16 * e4, col], vals)
                foff = lax.bitwise_and(ivec[l], 7) * 16
                fvals = buf_f[p, r, pl.ds(foff, 16)]
                plsc.store_scatter(obuf_f, [lanes, col], fvals)
        return ()

    lax.fori_loop(0, NCH, chunk, (), unroll=False)
    pltpu.sync_copy(obuf_i, ieT_hbm.at[:, pl.ds(base, BPW)])
    pltpu.sync_copy(obuf_f, feT_hbm.at[:, pl.ds(base, BPW)])


@functools.cache
def _rows():
    return pl.kernel(
        _rows_body,
        out_type=(
            jax.ShapeDtypeStruct((EMBED, B), jnp.float32),
            jax.ShapeDtypeStruct((FEAT, B), jnp.float32),
        ),
        mesh=plsc.VectorSubcoreMesh(
            core_axis_name="c", subcore_axis_name="s",
            num_cores=NC, num_subcores=NS),
        scratch_types=[
            pltpu.VMEM((BPW,), jnp.int32),
            pltpu.VMEM((2, CH, EMBED), jnp.float32),
            pltpu.VMEM((2, CH, 128), jnp.float32),
            pltpu.VMEM((EMBED, BPW), jnp.float32),
            pltpu.VMEM((FEAT, BPW), jnp.float32),
            pltpu.SemaphoreType.DMA,
        ],
        compiler_params=pltpu.CompilerParams(
            use_tc_tiling_on_sc=True, needs_layout_passes=False),
    )


BLK = 2048


def _mlp_body(uid, ueT, ieT, feT, tail, W1T, b1, W2T, b2, W3T, b3, WwT, bw,
              out):
    uid_r = uid[...]
    ueT_v = ueT[...]
    ieT_v = ieT[...]
    feT_v = feT[...]
    tail_off = uid_r - U_TAIL
    rows = lax.broadcasted_iota(jnp.int32, (EMBED, BLK), 0)
    ohT = (rows == tail_off).astype(jnp.float32)
    dg = lambda a, b: lax.dot_general(
        a, b, (((0,), (0,)), ((), ())),
        preferred_element_type=jnp.float32)
    corrT = dg(tail[...], ohT)
    ueT_u = jnp.where(uid_r >= U_TAIL, corrT, ueT_v)

    dot = functools.partial(jnp.dot, preferred_element_type=jnp.float32)
    W1T_v = W1T[...]
    WwT_v = WwT[...]
    h1 = dot(W1T_v[:, :EMBED], ueT_u) + dot(W1T_v[:, EMBED:2 * EMBED], ieT_v)
    h1 = h1 + dot(W1T_v[:, 2 * EMBED:], feT_v) + b1[...]
    h1 = jnp.maximum(h1, 0.0)
    h2 = jnp.maximum(dot(W2T[...], h1) + b2[...], 0.0)
    deep = jnp.maximum(dot(W3T[...], h2) + b3[...], 0.0)
    wide = (dot(WwT_v[:, :EMBED], ueT_u) + dot(WwT_v[:, EMBED:2 * EMBED], ieT_v)
            + dot(WwT_v[:, 2 * EMBED:], feT_v) + bw[...])
    out[...] = (deep + wide)[0, :]


def _mlp(uid_s, ueT, ieT, feT, tail, W1T, b1, W2T, b2, W3T, b3, WwT, bw):
    d_in = 2 * EMBED + FEAT
    grid = B // BLK
    cols = lambda i: (0, i)
    full = lambda i: (0, 0)
    return pl.pallas_call(
        _mlp_body,
        grid=(grid,),
        in_specs=[
            pl.BlockSpec((1, BLK), cols),
            pl.BlockSpec((EMBED, BLK), cols),
            pl.BlockSpec((EMBED, BLK), cols),
            pl.BlockSpec((FEAT, BLK), cols),
            pl.BlockSpec((EMBED, EMBED), full),
            pl.BlockSpec((64, d_in), full),
            pl.BlockSpec((64, 1), full),
            pl.BlockSpec((32, 64), full),
            pl.BlockSpec((32, 1), full),
            pl.BlockSpec((1, 32), full),
            pl.BlockSpec((1, 1), full),
            pl.BlockSpec((1, d_in), full),
            pl.BlockSpec((1, 1), full),
        ],
        out_specs=pl.BlockSpec((BLK,), lambda i: (i,)),
        out_shape=jax.ShapeDtypeStruct((B,), jnp.float32),
    )(uid_s, ueT, ieT, feT, tail, W1T, b1, W2T, b2, W3T, b3, WwT, bw)


def kernel(user_id, item_id, user_table, item_table, feat_table,
           W1, b1, W2, b2, W3, b3, Ww, bw):
    uid = user_id.astype(jnp.int32)
    iid = item_id.astype(jnp.int32)
    pos = lax.iota(jnp.int32, B)
    uid_s, iid_s, perm = lax.sort((uid, iid, pos), dimension=0, num_keys=1)
    tail = user_table[U_TAIL:, :]
    n_items = feat_table.shape[0]
    featp = feat_table.reshape(n_items * FEAT // 128, 128)
    ueT_s = _scan()(uid_s, user_table.T)
    ieT_s, feT_s = _rows()(iid_s, item_table, featp)
    res_s = _mlp(uid_s.reshape(1, B), ueT_s, ieT_s, feT_s, tail,
                 W1.T, b1.reshape(64, 1), W2.T, b2.reshape(32, 1),
                 W3.T, b3.reshape(1, 1), Ww.T, bw.reshape(1, 1))
    return lax.sort((perm, res_s), dimension=0, num_keys=1)[1]
